# baseline (speedup 1.0000x reference)
import functools

import jax
import jax.numpy as jnp
from jax import lax
from jax.experimental import pallas as pl
from jax.experimental.pallas import tpu as pltpu
from jax.experimental.pallas import tpu_sc as plsc

_N = 10000
_D = 128
_HD = 64
_E = 320000
_EPAD = 327680
_NTILES = 16
_EPT = _EPAD // _NTILES
_CHUNK = 128
_NCHUNKS = _EPT // _CHUNK
_NACC = 10112
_ZR = _NACC // _NTILES
_DNUMS = lax.GatherDimensionNumbers(offset_dims=(), collapsed_slice_dims=(0,),
                                    start_index_map=(0,))


def _prep_body(h_ref, wfcT_ref, bfc_ref, waS_ref, waD_ref, scal_ref,
               wpad_ref, ezs_ref, asrc_ref, adstb_ref, wc_ref):
    z = jnp.dot(h_ref[...], wfcT_ref[...],
                preferred_element_type=jnp.float32) + bfc_ref[...]
    gmax = jnp.max(z, axis=0, keepdims=True)
    ezs_ref[...] = jnp.exp(z - gmax)
    asrc_ref[...] = jnp.dot(z, waS_ref[...], preferred_element_type=jnp.float32)
    ad = jnp.dot(z, waD_ref[...], preferred_element_type=jnp.float32)
    adstb_ref[0:_N, :] = ad + scal_ref[0, 1]
    adstb_ref[_N:_NACC, :] = jnp.zeros((_NACC - _N, 1), jnp.float32)
    wc_ref[...] = wpad_ref[...] * scal_ref[0, 0]


_prep = pl.pallas_call(
    _prep_body,
    out_shape=[
        jax.ShapeDtypeStruct((_N, _D), jnp.float32),
        jax.ShapeDtypeStruct((_N, 1), jnp.float32),
        jax.ShapeDtypeStruct((_NACC, 1), jnp.float32),
        jax.ShapeDtypeStruct((_EPAD // _D, _D), jnp.float32),
    ],
    in_specs=[
        pl.BlockSpec(memory_space=pltpu.VMEM),
        pl.BlockSpec(memory_space=pltpu.VMEM),
        pl.BlockSpec(memory_space=pltpu.VMEM),
        pl.BlockSpec(memory_space=pltpu.VMEM),
        pl.BlockSpec(memory_space=pltpu.VMEM),
        pl.BlockSpec(memory_space=pltpu.SMEM),
        pl.BlockSpec(memory_space=pltpu.VMEM),
    ],
    out_specs=[
        pl.BlockSpec(memory_space=pltpu.VMEM),
        pl.BlockSpec(memory_space=pltpu.VMEM),
        pl.BlockSpec(memory_space=pltpu.VMEM),
        pl.BlockSpec(memory_space=pltpu.VMEM),
    ],
)


@functools.partial(
    pl.kernel,
    out_type=jax.ShapeDtypeStruct((2 * _NACC, _D), jnp.float32),
    mesh=plsc.VectorSubcoreMesh(core_axis_name="c", subcore_axis_name="s"),
    scratch_types=[
        pltpu.VMEM((_CHUNK,), jnp.int32),
        pltpu.VMEM((_CHUNK,), jnp.int32),
        pltpu.VMEM((_CHUNK,), jnp.float32),
        pltpu.VMEM((_CHUNK,), jnp.float32),
        pltpu.VMEM((_CHUNK,), jnp.float32),
        pltpu.VMEM((_CHUNK, _D), jnp.float32),
        pltpu.VMEM((_CHUNK,), jnp.int32),
        pltpu.VMEM((_CHUNK,), jnp.int32),
        pltpu.VMEM((_CHUNK,), jnp.float32),
        pltpu.VMEM((_CHUNK,), jnp.float32),
        pltpu.VMEM((_CHUNK,), jnp.float32),
        pltpu.VMEM((_CHUNK, _D), jnp.float32),
        pltpu.VMEM((8, _D), jnp.float32),
        pltpu.VMEM_SHARED((_NACC, _D), jnp.float32),
        pltpu.SemaphoreType.DMA,
        pltpu.SemaphoreType.DMA,
        pltpu.SemaphoreType.DMA,
        pltpu.SemaphoreType.DMA,
    ],
)
def _sc_edge(ezs, asrc, adstb, srcp, dstp, wcp, zer, out,
             isrcA, idstA, asgA, adgA, wcgA, rowsA,
             isrcB, idstB, asgB, adgB, wcgB, rowsB,
             fbuf, acc, semA, semB, isemA, isemB):
    cid = lax.axis_index("c")
    sid = lax.axis_index("s")
    hoff = cid * _HD
    moff = _HD - hoff
    pltpu.sync_copy(zer, acc.at[pl.ds(sid * _ZR, _ZR)])
    plsc.subcore_barrier()

    ebase = sid * _EPT
    A = (isrcA, idstA, asgA, adgA, wcgA, rowsA, semA, isemA)
    B = (isrcB, idstB, asgB, adgB, wcgB, rowsB, semB, isemB)

    def issue_idx(k, S):
        isrc, idst, asg, adg, wcg, rows, sem, isem = S
        base = ebase + k * _CHUNK
        pltpu.async_copy(srcp.at[pl.ds(base, _CHUNK)], isrc, isem)
        pltpu.async_copy(dstp.at[pl.ds(base, _CHUNK)], idst, isem)
        pltpu.async_copy(wcp.at[pl.ds(base, _CHUNK)], wcg, isem)

    def drain_idx(S):
        isrc, idst, asg, adg, wcg, rows, sem, isem = S
        pltpu.make_async_copy(srcp.at[pl.ds(0, _CHUNK)], isrc, isem).wait()
        pltpu.make_async_copy(dstp.at[pl.ds(0, _CHUNK)], idst, isem).wait()
        pltpu.make_async_copy(wcp.at[pl.ds(0, _CHUNK)], wcg, isem).wait()

    def issue(S):
        isrc, idst, asg, adg, wcg, rows, sem, isem = S
        pltpu.async_copy(asrc.at[isrc], asg, sem)
        pltpu.async_copy(adstb.at[idst], adg, sem)
        pltpu.async_copy(ezs.at[isrc], rows, sem)

    def drain(S):
        isrc, idst, asg, adg, wcg, rows, sem, isem = S
        pltpu.make_async_copy(asrc.at[pl.ds(0, _CHUNK)], asg, sem).wait()
        pltpu.make_async_copy(adstb.at[pl.ds(0, _CHUNK)], adg, sem).wait()
        pltpu.make_async_copy(ezs.at[pl.ds(0, _CHUNK)], rows, sem).wait()

    def compute(S):
        isrc, idst, asg, adg, wcg, rows, sem, isem = S
        for g in range(_CHUNK // 16):
            sl = pl.ds(g * 16, 16)
            t = asg[sl] + adg[sl] + wcg[sl]
            sv16 = jnp.maximum(t, t * 0.01)
            for j in range(16):
                e = g * 16 + j
                sp = lax.gather(sv16, jnp.full((16, 1), j, jnp.int32),
                                _DNUMS, (1,),
                                mode=lax.GatherScatterMode.PROMISE_IN_BOUNDS)
                for q in range(_HD // 16):
                    rv = rows[e, pl.ds(hoff + q * 16, 16)]
                    rows[e, pl.ds(moff + q * 16, 16)] = rv * sp
        pltpu.sync_copy(rows, acc.at[idst], add=True)

    issue_idx(0, A)
    drain_idx(A)
    issue(A)
    issue_idx(1, B)

    def body(m, carry):
        k0 = 2 * m
        last = m >= _NCHUNKS // 2 - 1
        drain_idx(B)
        issue(B)
        drain(A)
        compute(A)

        @pl.when(jnp.logical_not(last))
        def _():
            issue_idx(k0 + 2, A)

        @pl.when(jnp.logical_not(last))
        def _():
            drain_idx(A)
            issue(A)

        drain(B)
        compute(B)

        @pl.when(jnp.logical_not(last))
        def _():
            issue_idx(k0 + 3, B)

        return carry

    lax.fori_loop(0, _NCHUNKS // 2, body, 0)
    plsc.subcore_barrier()

    r0 = sid * _ZR

    def fin(i, carry):
        rb = r0 + i * 8
        pltpu.sync_copy(acc.at[pl.ds(rb, 8)], fbuf)
        for r in range(8):
            for q in range(_HD // 16):
                n = fbuf[r, pl.ds(moff + q * 16, 16)]
                d = fbuf[r, pl.ds(hoff + q * 16, 16)]
                fbuf[r, pl.ds(moff + q * 16, 16)] = n / jnp.where(d > 0.0, d, 1.0)
        pltpu.sync_copy(fbuf, out.at[pl.ds(cid * _NACC + rb, 8)])
        return carry

    lax.fori_loop(0, _ZR // 8, fin, 0)


def kernel(h, edge_index, edge_weight, W_fc, b_fc, W_att, b_att):
    h2 = h[0]
    src = edge_index[0].astype(jnp.int32)
    dst = edge_index[1].astype(jnp.int32)
    ew = edge_weight[:, 0].astype(jnp.float32)
    npad = _EPAD - _E
    pi = jnp.arange(npad, dtype=jnp.int32) % 16
    srcp = jnp.concatenate([src, pi])
    dstp = jnp.concatenate([dst, _N + pi])
    wp = jnp.concatenate([ew, jnp.zeros((npad,), jnp.float32)])

    wfcT = W_fc.T
    bfc = b_fc.reshape(1, _D)
    waS = W_att[0, :_D].reshape(_D, 1)
    waD = W_att[0, _D:2 * _D].reshape(_D, 1)
    scal = jnp.stack([W_att[0, 2 * _D], b_att[0]]).reshape(1, 2)

    ezs, asrc, adstb, wc = _prep(h2, wfcT, bfc, waS, waD, scal,
                                 wp.reshape(_EPAD // _D, _D))

    zer = jnp.zeros((_ZR, _D), jnp.float32)
    outs = _sc_edge(ezs, asrc.reshape(_N), adstb.reshape(_NACC),
                    srcp, dstp, wc.reshape(_EPAD), zer)

    return jnp.concatenate([outs[:_N, _HD:], outs[_NACC:_NACC + _N, :_HD]],
                           axis=1)[None]


# speedup vs baseline: 2.3181x; 2.3181x over previous
import functools

import jax
import jax.numpy as jnp
from jax import lax
from jax.experimental import pallas as pl
from jax.experimental.pallas import tpu as pltpu
from jax.experimental.pallas import tpu_sc as plsc

_N = 10000
_D = 128
_HD = 64
_E = 320000
_EPAD = 327680
_NTILES = 16
_EPT = _EPAD // _NTILES
_CHUNK = 128
_NCHUNKS = _EPT // _CHUNK
_NACC = 10112
_ZR = _NACC // _NTILES
_DNUMS = lax.GatherDimensionNumbers(offset_dims=(), collapsed_slice_dims=(0,),
                                    start_index_map=(0,))


def _prep_body(h_ref, wfcT_ref, bfc_ref, waS_ref, waD_ref, scal_ref,
               wpad_ref, ezs_ref, asrc_ref, adstb_ref, wc_ref):
    z = jnp.dot(h_ref[...], wfcT_ref[...],
                preferred_element_type=jnp.float32) + bfc_ref[...]
    gmax = jnp.max(z, axis=0, keepdims=True)
    ezs_ref[...] = jnp.exp(z - gmax)
    asrc_ref[...] = jnp.dot(z, waS_ref[...], preferred_element_type=jnp.float32)
    ad = jnp.dot(z, waD_ref[...], preferred_element_type=jnp.float32)
    adstb_ref[0:_N, :] = ad + scal_ref[0, 1]
    adstb_ref[_N:_NACC, :] = jnp.zeros((_NACC - _N, 1), jnp.float32)
    wc_ref[...] = wpad_ref[...] * scal_ref[0, 0]


_prep = pl.pallas_call(
    _prep_body,
    out_shape=[
        jax.ShapeDtypeStruct((_N, _D), jnp.float32),
        jax.ShapeDtypeStruct((_N, 1), jnp.float32),
        jax.ShapeDtypeStruct((_NACC, 1), jnp.float32),
        jax.ShapeDtypeStruct((_EPAD // _D, _D), jnp.float32),
    ],
    in_specs=[
        pl.BlockSpec(memory_space=pltpu.VMEM),
        pl.BlockSpec(memory_space=pltpu.VMEM),
        pl.BlockSpec(memory_space=pltpu.VMEM),
        pl.BlockSpec(memory_space=pltpu.VMEM),
        pl.BlockSpec(memory_space=pltpu.VMEM),
        pl.BlockSpec(memory_space=pltpu.SMEM),
        pl.BlockSpec(memory_space=pltpu.VMEM),
    ],
    out_specs=[
        pl.BlockSpec(memory_space=pltpu.VMEM),
        pl.BlockSpec(memory_space=pltpu.VMEM),
        pl.BlockSpec(memory_space=pltpu.VMEM),
        pl.BlockSpec(memory_space=pltpu.VMEM),
    ],
)


@functools.partial(
    pl.kernel,
    out_type=jax.ShapeDtypeStruct((2 * _NACC, _D), jnp.float32),
    mesh=plsc.VectorSubcoreMesh(core_axis_name="c", subcore_axis_name="s"),
    scratch_types=[
        pltpu.VMEM((_CHUNK,), jnp.int32),
        pltpu.VMEM((_CHUNK,), jnp.int32),
        pltpu.VMEM((_CHUNK,), jnp.float32),
        pltpu.VMEM((_CHUNK,), jnp.float32),
        pltpu.VMEM((_CHUNK,), jnp.float32),
        pltpu.VMEM((_CHUNK, _D), jnp.float32),
        pltpu.VMEM((_CHUNK,), jnp.int32),
        pltpu.VMEM((_CHUNK,), jnp.int32),
        pltpu.VMEM((_CHUNK,), jnp.float32),
        pltpu.VMEM((_CHUNK,), jnp.float32),
        pltpu.VMEM((_CHUNK,), jnp.float32),
        pltpu.VMEM((_CHUNK, _D), jnp.float32),
        pltpu.VMEM((8, _D), jnp.float32),
        pltpu.VMEM_SHARED((_NACC, _D), jnp.float32),
        pltpu.SemaphoreType.DMA,
        pltpu.SemaphoreType.DMA,
        pltpu.SemaphoreType.DMA,
        pltpu.SemaphoreType.DMA,
    ],
)
def _sc_edge(ezs, asrc, adstb, srcp, dstp, wcp, zer, out,
             isrcA, idstA, asgA, adgA, wcgA, rowsA,
             isrcB, idstB, asgB, adgB, wcgB, rowsB,
             fbuf, acc, semA, semB, isemA, isemB):
    cid = lax.axis_index("c")
    sid = lax.axis_index("s")
    hoff = cid * _HD
    moff = _HD - hoff
    pltpu.sync_copy(zer, acc.at[pl.ds(sid * _ZR, _ZR)])
    plsc.subcore_barrier()

    ebase = sid * _EPT
    A = (isrcA, idstA, asgA, adgA, wcgA, rowsA, semA, isemA)
    B = (isrcB, idstB, asgB, adgB, wcgB, rowsB, semB, isemB)

    def issue_idx(k, S):
        isrc, idst, asg, adg, wcg, rows, sem, isem = S
        base = ebase + k * _CHUNK
        pltpu.async_copy(srcp.at[pl.ds(base, _CHUNK)], isrc, isem)
        pltpu.async_copy(dstp.at[pl.ds(base, _CHUNK)], idst, isem)
        pltpu.async_copy(wcp.at[pl.ds(base, _CHUNK)], wcg, isem)

    def drain_idx(S):
        isrc, idst, asg, adg, wcg, rows, sem, isem = S
        pltpu.make_async_copy(srcp.at[pl.ds(0, _CHUNK)], isrc, isem).wait()
        pltpu.make_async_copy(dstp.at[pl.ds(0, _CHUNK)], idst, isem).wait()
        pltpu.make_async_copy(wcp.at[pl.ds(0, _CHUNK)], wcg, isem).wait()

    def issue(S):
        isrc, idst, asg, adg, wcg, rows, sem, isem = S
        pltpu.async_copy(asrc.at[isrc], asg, sem)
        pltpu.async_copy(adstb.at[idst], adg, sem)
        pltpu.async_copy(ezs.at[isrc], rows, sem)

    def drain(S):
        isrc, idst, asg, adg, wcg, rows, sem, isem = S
        pltpu.make_async_copy(asrc.at[pl.ds(0, _CHUNK)], asg, sem).wait()
        pltpu.make_async_copy(adstb.at[pl.ds(0, _CHUNK)], adg, sem).wait()
        pltpu.make_async_copy(ezs.at[pl.ds(0, _CHUNK)], rows, sem).wait()

    def compute(S):
        isrc, idst, asg, adg, wcg, rows, sem, isem = S

        def cgrp(h, m):
            def gbody(g, carry):
                sl = pl.ds(g * 16, 16)
                t = asg[sl] + adg[sl] + wcg[sl]
                sv16 = jnp.maximum(t, t * 0.01)
                ge = g * 16
                for j in range(16):
                    sp = lax.gather(sv16, jnp.full((16, 1), j, jnp.int32),
                                    _DNUMS, (1,),
                                    mode=lax.GatherScatterMode.PROMISE_IN_BOUNDS)
                    for q in range(_HD // 16):
                        rv = rows[ge + j, pl.ds(h + q * 16, 16)]
                        rows[ge + j, pl.ds(m + q * 16, 16)] = rv * sp
                return carry
            lax.fori_loop(0, _CHUNK // 16, gbody, 0)

        @pl.when(cid == 0)
        def _():
            cgrp(0, _HD)

        @pl.when(cid == 1)
        def _():
            cgrp(_HD, 0)

        pltpu.sync_copy(rows, acc.at[idst], add=True)

    issue_idx(0, A)
    drain_idx(A)
    issue(A)
    issue_idx(1, B)

    def body(m, carry):
        k0 = 2 * m
        last = m >= _NCHUNKS // 2 - 1
        drain_idx(B)
        issue(B)
        drain(A)
        compute(A)

        @pl.when(jnp.logical_not(last))
        def _():
            issue_idx(k0 + 2, A)

        @pl.when(jnp.logical_not(last))
        def _():
            drain_idx(A)
            issue(A)

        drain(B)
        compute(B)

        @pl.when(jnp.logical_not(last))
        def _():
            issue_idx(k0 + 3, B)

        return carry

    lax.fori_loop(0, _NCHUNKS // 2, body, 0)
    plsc.subcore_barrier()

    r0 = sid * _ZR

    def fin(i, carry):
        rb = r0 + i * 8
        pltpu.sync_copy(acc.at[pl.ds(rb, 8)], fbuf)
        for r in range(8):
            for q in range(_HD // 16):
                n = fbuf[r, pl.ds(moff + q * 16, 16)]
                d = fbuf[r, pl.ds(hoff + q * 16, 16)]
                fbuf[r, pl.ds(moff + q * 16, 16)] = n / jnp.where(d > 0.0, d, 1.0)
        pltpu.sync_copy(fbuf, out.at[pl.ds(cid * _NACC + rb, 8)])
        return carry

    lax.fori_loop(0, _ZR // 8, fin, 0)


def kernel(h, edge_index, edge_weight, W_fc, b_fc, W_att, b_att):
    h2 = h[0]
    src = edge_index[0].astype(jnp.int32)
    dst = edge_index[1].astype(jnp.int32)
    ew = edge_weight[:, 0].astype(jnp.float32)
    npad = _EPAD - _E
    pi = jnp.arange(npad, dtype=jnp.int32) % 16
    srcp = jnp.concatenate([src, pi])
    dstp = jnp.concatenate([dst, _N + pi])
    wp = jnp.concatenate([ew, jnp.zeros((npad,), jnp.float32)])

    wfcT = W_fc.T
    bfc = b_fc.reshape(1, _D)
    waS = W_att[0, :_D].reshape(_D, 1)
    waD = W_att[0, _D:2 * _D].reshape(_D, 1)
    scal = jnp.stack([W_att[0, 2 * _D], b_att[0]]).reshape(1, 2)

    ezs, asrc, adstb, wc = _prep(h2, wfcT, bfc, waS, waD, scal,
                                 wp.reshape(_EPAD // _D, _D))

    zer = jnp.zeros((_ZR, _D), jnp.float32)
    outs = _sc_edge(ezs, asrc.reshape(_N), adstb.reshape(_NACC),
                    srcp, dstp, wc.reshape(_EPAD), zer)

    return jnp.concatenate([outs[:_N, _HD:], outs[_NACC:_NACC + _N, :_HD]],
                           axis=1)[None]


# speedup vs baseline: 2.3222x; 1.0018x over previous
import functools

import jax
import jax.numpy as jnp
from jax import lax
from jax.experimental import pallas as pl
from jax.experimental.pallas import tpu as pltpu
from jax.experimental.pallas import tpu_sc as plsc

_N = 10000
_D = 128
_HD = 64
_E = 320000
_EPAD = 327680
_NTILES = 16
_EPT = _EPAD // _NTILES
_CHUNK = 128
_NCHUNKS = _EPT // _CHUNK
_NACC = 10112
_ZR = _NACC // _NTILES
_DNUMS = lax.GatherDimensionNumbers(offset_dims=(), collapsed_slice_dims=(0,),
                                    start_index_map=(0,))


_ER = _E // _D
_EPR = _EPAD // _D


def _prep_body(h_ref, wfcT_ref, bfc_ref, waS_ref, waD_ref, scal_ref,
               ew_ref, ei0_ref, ei1_ref,
               ezs_ref, asrc_ref, adstb_ref, wc_ref, srcp_ref, dstp_ref):
    z = jnp.dot(h_ref[...], wfcT_ref[...],
                preferred_element_type=jnp.float32) + bfc_ref[...]
    gmax = jnp.max(z, axis=0, keepdims=True)
    ezs_ref[...] = jnp.exp(z - gmax)
    asrc_ref[...] = jnp.dot(z, waS_ref[...], preferred_element_type=jnp.float32)
    ad = jnp.dot(z, waD_ref[...], preferred_element_type=jnp.float32)
    adstb_ref[0:_N, :] = ad + scal_ref[0, 1]
    adstb_ref[_N:_NACC, :] = jnp.zeros((_NACC - _N, 1), jnp.float32)
    wc_ref[0:_ER, :] = ew_ref[...] * scal_ref[0, 0]
    wc_ref[_ER:_EPR, :] = jnp.zeros((_EPR - _ER, _D), jnp.float32)
    pad = lax.broadcasted_iota(jnp.int32, (_EPR - _ER, _D), 1) % 16
    srcp_ref[0:_ER, :] = ei0_ref[...]
    srcp_ref[_ER:_EPR, :] = pad
    dstp_ref[0:_ER, :] = ei1_ref[...]
    dstp_ref[_ER:_EPR, :] = pad + _N


_prep = pl.pallas_call(
    _prep_body,
    out_shape=[
        jax.ShapeDtypeStruct((_N, _D), jnp.float32),
        jax.ShapeDtypeStruct((_N, 1), jnp.float32),
        jax.ShapeDtypeStruct((_NACC, 1), jnp.float32),
        jax.ShapeDtypeStruct((_EPAD // _D, _D), jnp.float32),
        jax.ShapeDtypeStruct((_EPAD // _D, _D), jnp.int32),
        jax.ShapeDtypeStruct((_EPAD // _D, _D), jnp.int32),
    ],
    in_specs=[pl.BlockSpec(memory_space=pltpu.VMEM)] * 5
             + [pl.BlockSpec(memory_space=pltpu.SMEM)]
             + [pl.BlockSpec(memory_space=pltpu.VMEM)] * 3,
    out_specs=[pl.BlockSpec(memory_space=pltpu.VMEM)] * 6,
)


@functools.partial(
    pl.kernel,
    out_type=jax.ShapeDtypeStruct((2 * _NACC, _D), jnp.float32),
    mesh=plsc.VectorSubcoreMesh(core_axis_name="c", subcore_axis_name="s"),
    scratch_types=[
        pltpu.VMEM((_CHUNK,), jnp.int32),
        pltpu.VMEM((_CHUNK,), jnp.int32),
        pltpu.VMEM((_CHUNK,), jnp.float32),
        pltpu.VMEM((_CHUNK,), jnp.float32),
        pltpu.VMEM((_CHUNK,), jnp.float32),
        pltpu.VMEM((_CHUNK, _D), jnp.float32),
        pltpu.VMEM((_CHUNK,), jnp.int32),
        pltpu.VMEM((_CHUNK,), jnp.int32),
        pltpu.VMEM((_CHUNK,), jnp.float32),
        pltpu.VMEM((_CHUNK,), jnp.float32),
        pltpu.VMEM((_CHUNK,), jnp.float32),
        pltpu.VMEM((_CHUNK, _D), jnp.float32),
        pltpu.VMEM((8, _D), jnp.float32),
        pltpu.VMEM_SHARED((_NACC, _D), jnp.float32),
        pltpu.SemaphoreType.DMA,
        pltpu.SemaphoreType.DMA,
        pltpu.SemaphoreType.DMA,
        pltpu.SemaphoreType.DMA,
    ],
)
def _sc_edge(ezs, asrc, adstb, srcp, dstp, wcp, zer, out,
             isrcA, idstA, asgA, adgA, wcgA, rowsA,
             isrcB, idstB, asgB, adgB, wcgB, rowsB,
             fbuf, acc, semA, semB, isemA, isemB):
    cid = lax.axis_index("c")
    sid = lax.axis_index("s")
    hoff = cid * _HD
    moff = _HD - hoff
    pltpu.sync_copy(zer, acc.at[pl.ds(sid * _ZR, _ZR)])
    plsc.subcore_barrier()

    ebase = sid * _EPT
    A = (isrcA, idstA, asgA, adgA, wcgA, rowsA, semA, isemA)
    B = (isrcB, idstB, asgB, adgB, wcgB, rowsB, semB, isemB)

    def issue_idx(k, S):
        isrc, idst, asg, adg, wcg, rows, sem, isem = S
        base = ebase + k * _CHUNK
        pltpu.async_copy(srcp.at[pl.ds(base, _CHUNK)], isrc, isem)
        pltpu.async_copy(dstp.at[pl.ds(base, _CHUNK)], idst, isem)
        pltpu.async_copy(wcp.at[pl.ds(base, _CHUNK)], wcg, isem)

    def drain_idx(S):
        isrc, idst, asg, adg, wcg, rows, sem, isem = S
        pltpu.make_async_copy(srcp.at[pl.ds(0, _CHUNK)], isrc, isem).wait()
        pltpu.make_async_copy(dstp.at[pl.ds(0, _CHUNK)], idst, isem).wait()
        pltpu.make_async_copy(wcp.at[pl.ds(0, _CHUNK)], wcg, isem).wait()

    def issue(S):
        isrc, idst, asg, adg, wcg, rows, sem, isem = S
        pltpu.async_copy(asrc.at[isrc], asg, sem)
        pltpu.async_copy(adstb.at[idst], adg, sem)
        pltpu.async_copy(ezs.at[isrc], rows, sem)

    def drain(S):
        isrc, idst, asg, adg, wcg, rows, sem, isem = S
        pltpu.make_async_copy(asrc.at[pl.ds(0, _CHUNK)], asg, sem).wait()
        pltpu.make_async_copy(adstb.at[pl.ds(0, _CHUNK)], adg, sem).wait()
        pltpu.make_async_copy(ezs.at[pl.ds(0, _CHUNK)], rows, sem).wait()

    def compute(S):
        isrc, idst, asg, adg, wcg, rows, sem, isem = S

        def cgrp(h, m):
            def gbody(g, carry):
                sl = pl.ds(g * 16, 16)
                t = asg[sl] + adg[sl] + wcg[sl]
                sv16 = jnp.maximum(t, t * 0.01)
                ge = g * 16
                for j in range(16):
                    sp = lax.gather(sv16, jnp.full((16, 1), j, jnp.int32),
                                    _DNUMS, (1,),
                                    mode=lax.GatherScatterMode.PROMISE_IN_BOUNDS)
                    for q in range(_HD // 16):
                        rv = rows[ge + j, pl.ds(h + q * 16, 16)]
                        rows[ge + j, pl.ds(m + q * 16, 16)] = rv * sp
                return carry
            lax.fori_loop(0, _CHUNK // 16, gbody, 0)

        @pl.when(cid == 0)
        def _():
            cgrp(0, _HD)

        @pl.when(cid == 1)
        def _():
            cgrp(_HD, 0)

        pltpu.sync_copy(rows, acc.at[idst], add=True)

    issue_idx(0, A)
    drain_idx(A)
    issue(A)
    issue_idx(1, B)

    def body(m, carry):
        k0 = 2 * m
        last = m >= _NCHUNKS // 2 - 1
        drain_idx(B)
        issue(B)
        drain(A)
        compute(A)

        @pl.when(jnp.logical_not(last))
        def _():
            issue_idx(k0 + 2, A)

        @pl.when(jnp.logical_not(last))
        def _():
            drain_idx(A)
            issue(A)

        drain(B)
        compute(B)

        @pl.when(jnp.logical_not(last))
        def _():
            issue_idx(k0 + 3, B)

        return carry

    lax.fori_loop(0, _NCHUNKS // 2, body, 0)
    plsc.subcore_barrier()

    r0 = sid * _ZR

    def fin(i, carry):
        rb = r0 + i * 8
        pltpu.sync_copy(acc.at[pl.ds(rb, 8)], fbuf)
        for r in range(8):
            for q in range(_HD // 16):
                n = fbuf[r, pl.ds(moff + q * 16, 16)]
                d = fbuf[r, pl.ds(hoff + q * 16, 16)]
                fbuf[r, pl.ds(moff + q * 16, 16)] = n / jnp.where(d > 0.0, d, 1.0)
        pltpu.sync_copy(fbuf, out.at[pl.ds(cid * _NACC + rb, 8)])
        return carry

    lax.fori_loop(0, _ZR // 8, fin, 0)


def kernel(h, edge_index, edge_weight, W_fc, b_fc, W_att, b_att):
    h2 = h[0]
    ei = edge_index.astype(jnp.int32)
    ew2d = edge_weight.astype(jnp.float32).reshape(_ER, _D)

    wfcT = W_fc.T
    bfc = b_fc.reshape(1, _D)
    waS = W_att[0, :_D].reshape(_D, 1)
    waD = W_att[0, _D:2 * _D].reshape(_D, 1)
    scal = jnp.stack([W_att[0, 2 * _D], b_att[0]]).reshape(1, 2)

    ezs, asrc, adstb, wc, srcp, dstp = _prep(
        h2, wfcT, bfc, waS, waD, scal, ew2d,
        ei[0].reshape(_ER, _D), ei[1].reshape(_ER, _D))

    zer = jnp.zeros((_ZR, _D), jnp.float32)
    outs = _sc_edge(ezs, asrc.reshape(_N), adstb.reshape(_NACC),
                    srcp.reshape(_EPAD), dstp.reshape(_EPAD),
                    wc.reshape(_EPAD), zer)

    return jnp.concatenate([outs[:_N, _HD:], outs[_NACC:_NACC + _N, :_HD]],
                           axis=1)[None]


# speedup vs baseline: 2.4747x; 1.0657x over previous
import functools

import jax
import jax.numpy as jnp
from jax import lax
from jax.experimental import pallas as pl
from jax.experimental.pallas import tpu as pltpu
from jax.experimental.pallas import tpu_sc as plsc

_N = 10000
_D = 128
_HD = 64
_E = 320000
_EPAD = 327680
_NTILES = 16
_EPT = _EPAD // _NTILES
_CHUNK = 128
_NCHUNKS = _EPT // _CHUNK
_NACC = 10112
_ZR = _NACC // _NTILES
_DNUMS = lax.GatherDimensionNumbers(offset_dims=(), collapsed_slice_dims=(0,),
                                    start_index_map=(0,))


_ER = _E // _D
_EPR = _EPAD // _D


def _prep_body(h_ref, wfcT_ref, bfc_ref, waS_ref, waD_ref, scal_ref,
               ew_ref, ei0_ref, ei1_ref,
               ezs_ref, asrc_ref, adstb_ref, wc_ref, srcp_ref, dstp_ref):
    z = jnp.dot(h_ref[...], wfcT_ref[...],
                preferred_element_type=jnp.float32) + bfc_ref[...]
    gmax = jnp.max(z, axis=0, keepdims=True)
    ezs_ref[...] = jnp.exp(z - gmax)
    asrc_ref[...] = jnp.dot(z, waS_ref[...], preferred_element_type=jnp.float32)
    ad = jnp.dot(z, waD_ref[...], preferred_element_type=jnp.float32)
    adstb_ref[0:_N, :] = ad + scal_ref[0, 1]
    adstb_ref[_N:_NACC, :] = jnp.zeros((_NACC - _N, 1), jnp.float32)
    wc_ref[0:_ER, :] = ew_ref[...] * scal_ref[0, 0]
    wc_ref[_ER:_EPR, :] = jnp.zeros((_EPR - _ER, _D), jnp.float32)
    pad = lax.broadcasted_iota(jnp.int32, (_EPR - _ER, _D), 1) % 16
    srcp_ref[0:_ER, :] = ei0_ref[...]
    srcp_ref[_ER:_EPR, :] = pad
    dstp_ref[0:_ER, :] = ei1_ref[...]
    dstp_ref[_ER:_EPR, :] = pad + _N


_prep = pl.pallas_call(
    _prep_body,
    out_shape=[
        jax.ShapeDtypeStruct((_N, _D), jnp.float32),
        jax.ShapeDtypeStruct((_N, 1), jnp.float32),
        jax.ShapeDtypeStruct((_NACC, 1), jnp.float32),
        jax.ShapeDtypeStruct((_EPAD // _D, _D), jnp.float32),
        jax.ShapeDtypeStruct((_EPAD // _D, _D), jnp.int32),
        jax.ShapeDtypeStruct((_EPAD // _D, _D), jnp.int32),
    ],
    in_specs=[pl.BlockSpec(memory_space=pltpu.VMEM)] * 5
             + [pl.BlockSpec(memory_space=pltpu.SMEM)]
             + [pl.BlockSpec(memory_space=pltpu.VMEM)] * 3,
    out_specs=[pl.BlockSpec(memory_space=pltpu.VMEM)] * 6,
)


@functools.partial(
    pl.kernel,
    out_type=jax.ShapeDtypeStruct((2 * _NACC, _D), jnp.float32),
    mesh=plsc.VectorSubcoreMesh(core_axis_name="c", subcore_axis_name="s"),
    scratch_types=[
        pltpu.VMEM((_CHUNK,), jnp.int32),
        pltpu.VMEM((_CHUNK,), jnp.int32),
        pltpu.VMEM((_CHUNK,), jnp.float32),
        pltpu.VMEM((_CHUNK,), jnp.float32),
        pltpu.VMEM((_CHUNK,), jnp.float32),
        pltpu.VMEM((_CHUNK, _D), jnp.float32),
        pltpu.VMEM((_CHUNK,), jnp.int32),
        pltpu.VMEM((_CHUNK,), jnp.int32),
        pltpu.VMEM((_CHUNK,), jnp.float32),
        pltpu.VMEM((_CHUNK,), jnp.float32),
        pltpu.VMEM((_CHUNK,), jnp.float32),
        pltpu.VMEM((_CHUNK, _D), jnp.float32),
        pltpu.VMEM((8, _D), jnp.float32),
        pltpu.VMEM((_CHUNK,), jnp.int32),
        pltpu.VMEM((_CHUNK,), jnp.int32),
        pltpu.VMEM_SHARED((_NACC, _D), jnp.float32),
        pltpu.SemaphoreType.DMA,
        pltpu.SemaphoreType.DMA,
        pltpu.SemaphoreType.DMA,
        pltpu.SemaphoreType.DMA,
        pltpu.SemaphoreType.DMA,
        pltpu.SemaphoreType.DMA,
    ],
)
def _sc_edge(ezs, asrc, adstb, srcp, dstp, wcp, zer, out,
             isrcA, idstA, asgA, adgA, wcgA, rowsA,
             isrcB, idstB, asgB, adgB, wcgB, rowsB,
             fbuf, sidxA, sidxB, acc, semA, semB, isemA, isemB, ssemA, ssemB):
    cid = lax.axis_index("c")
    sid = lax.axis_index("s")
    hoff = cid * _HD
    moff = _HD - hoff
    pltpu.sync_copy(zer, acc.at[pl.ds(sid * _ZR, _ZR)])
    plsc.subcore_barrier()

    ebase = sid * _EPT
    A = (isrcA, idstA, asgA, adgA, wcgA, rowsA, semA, isemA, sidxA, ssemA)
    B = (isrcB, idstB, asgB, adgB, wcgB, rowsB, semB, isemB, sidxB, ssemB)

    def issue_idx(k, S):
        isrc, idst, asg, adg, wcg, rows, sem, isem, sidx, ssem = S
        base = ebase + k * _CHUNK
        pltpu.async_copy(srcp.at[pl.ds(base, _CHUNK)], isrc, isem)
        pltpu.async_copy(dstp.at[pl.ds(base, _CHUNK)], idst, isem)
        pltpu.async_copy(wcp.at[pl.ds(base, _CHUNK)], wcg, isem)

    def drain_idx(S):
        isrc, idst, asg, adg, wcg, rows, sem, isem, sidx, ssem = S
        pltpu.make_async_copy(srcp.at[pl.ds(0, _CHUNK)], isrc, isem).wait()
        pltpu.make_async_copy(dstp.at[pl.ds(0, _CHUNK)], idst, isem).wait()
        pltpu.make_async_copy(wcp.at[pl.ds(0, _CHUNK)], wcg, isem).wait()

    def issue(S):
        isrc, idst, asg, adg, wcg, rows, sem, isem, sidx, ssem = S
        pltpu.async_copy(asrc.at[isrc], asg, sem)
        pltpu.async_copy(adstb.at[idst], adg, sem)
        pltpu.async_copy(ezs.at[isrc], rows, sem)

    def drain(S):
        isrc, idst, asg, adg, wcg, rows, sem, isem, sidx, ssem = S
        pltpu.make_async_copy(asrc.at[pl.ds(0, _CHUNK)], asg, sem).wait()
        pltpu.make_async_copy(adstb.at[pl.ds(0, _CHUNK)], adg, sem).wait()
        pltpu.make_async_copy(ezs.at[pl.ds(0, _CHUNK)], rows, sem).wait()

    def compute(S):
        isrc, idst, asg, adg, wcg, rows, sem, isem, sidx, ssem = S

        def cgrp(h, m):
            def gbody(g, carry):
                sl = pl.ds(g * 16, 16)
                t = asg[sl] + adg[sl] + wcg[sl]
                sv16 = jnp.maximum(t, t * 0.01)
                ge = g * 16
                for j in range(16):
                    sp = lax.gather(sv16, jnp.full((16, 1), j, jnp.int32),
                                    _DNUMS, (1,),
                                    mode=lax.GatherScatterMode.PROMISE_IN_BOUNDS)
                    for q in range(_HD // 16):
                        rv = rows[ge + j, pl.ds(h + q * 16, 16)]
                        rows[ge + j, pl.ds(m + q * 16, 16)] = rv * sp
                return carry
            lax.fori_loop(0, _CHUNK // 16, gbody, 0)

        @pl.when(cid == 0)
        def _():
            cgrp(0, _HD)

        @pl.when(cid == 1)
        def _():
            cgrp(_HD, 0)

        for v in range(_CHUNK // 16):
            sl = pl.ds(v * 16, 16)
            sidx[sl] = idst[sl]
        pltpu.async_copy(rows, acc.at[sidx], ssem, add=True)

    def drain_sc(S):
        isrc, idst, asg, adg, wcg, rows, sem, isem, sidx, ssem = S
        pltpu.make_async_copy(rows, acc.at[sidx], ssem).wait()

    issue_idx(0, A)
    drain_idx(A)
    issue(A)
    issue_idx(1, B)

    def body(m, carry):
        k0 = 2 * m
        last = m >= _NCHUNKS // 2 - 1
        drain_idx(B)

        @pl.when(m > 0)
        def _():
            drain_sc(B)

        issue(B)
        drain(A)
        compute(A)

        @pl.when(jnp.logical_not(last))
        def _():
            issue_idx(k0 + 2, A)

        @pl.when(jnp.logical_not(last))
        def _():
            drain_idx(A)
            drain_sc(A)
            issue(A)

        drain(B)
        compute(B)

        @pl.when(jnp.logical_not(last))
        def _():
            issue_idx(k0 + 3, B)

        return carry

    lax.fori_loop(0, _NCHUNKS // 2, body, 0)
    drain_sc(A)
    drain_sc(B)
    plsc.subcore_barrier()

    r0 = sid * _ZR

    def fin(i, carry):
        rb = r0 + i * 8
        pltpu.sync_copy(acc.at[pl.ds(rb, 8)], fbuf)
        for r in range(8):
            for q in range(_HD // 16):
                n = fbuf[r, pl.ds(moff + q * 16, 16)]
                d = fbuf[r, pl.ds(hoff + q * 16, 16)]
                fbuf[r, pl.ds(moff + q * 16, 16)] = n / jnp.where(d > 0.0, d, 1.0)
        pltpu.sync_copy(fbuf, out.at[pl.ds(cid * _NACC + rb, 8)])
        return carry

    lax.fori_loop(0, _ZR // 8, fin, 0)


def kernel(h, edge_index, edge_weight, W_fc, b_fc, W_att, b_att):
    h2 = h[0]
    ei = edge_index.astype(jnp.int32)
    ew2d = edge_weight.astype(jnp.float32).reshape(_ER, _D)

    wfcT = W_fc.T
    bfc = b_fc.reshape(1, _D)
    waS = W_att[0, :_D].reshape(_D, 1)
    waD = W_att[0, _D:2 * _D].reshape(_D, 1)
    scal = jnp.stack([W_att[0, 2 * _D], b_att[0]]).reshape(1, 2)

    ezs, asrc, adstb, wc, srcp, dstp = _prep(
        h2, wfcT, bfc, waS, waD, scal, ew2d,
        ei[0].reshape(_ER, _D), ei[1].reshape(_ER, _D))

    zer = jnp.zeros((_ZR, _D), jnp.float32)
    outs = _sc_edge(ezs, asrc.reshape(_N), adstb.reshape(_NACC),
                    srcp.reshape(_EPAD), dstp.reshape(_EPAD),
                    wc.reshape(_EPAD), zer)

    return jnp.concatenate([outs[:_N, _HD:], outs[_NACC:_NACC + _N, :_HD]],
                           axis=1)[None]
